# baseline (speedup 1.0000x reference)
import jax
import jax.numpy as jnp
from jax.experimental import pallas as pl
from jax.experimental.pallas import tpu as pltpu

_V = 200
_BLOCK_B = 32


def _fused_kernel(idx_ref, tgt_ref, emb_ref, logits_ref, losssum_ref):
    emb = emb_ref[...]
    rb, T = idx_ref.shape
    n = rb * T
    v_pad = emb.shape[1]

    idx_blk = idx_ref[...].astype(jnp.float32)
    tgt_blk = tgt_ref[...].astype(jnp.float32)

    row = jax.lax.broadcasted_iota(jnp.int32, (n, 1), 0)
    colb = jax.lax.broadcasted_iota(jnp.int32, (n, rb), 1)
    sel = ((row // T) == colb).astype(jnp.float32)
    colt = jax.lax.broadcasted_iota(jnp.int32, (n, T), 1)
    tmask = colt == (row % T)

    rows_idx = jnp.dot(sel, idx_blk, preferred_element_type=jnp.float32)
    idx_i = jnp.sum(jnp.where(tmask, rows_idx, 0.0),
                    axis=1, keepdims=True).astype(jnp.int32)
    rows_tgt = jnp.dot(sel, tgt_blk, preferred_element_type=jnp.float32)
    tgt_i = jnp.sum(jnp.where(tmask, rows_tgt, 0.0),
                    axis=1, keepdims=True).astype(jnp.int32)

    colv = jax.lax.broadcasted_iota(jnp.int32, (n, v_pad), 1)
    onehot = (colv == idx_i).astype(jnp.float32)
    logits = jnp.dot(onehot, emb, preferred_element_type=jnp.float32)
    logits_ref[...] = logits

    m = jnp.max(emb, axis=1, keepdims=True)
    lse_vec = m + jnp.log(jnp.sum(jnp.exp(emb - m), axis=1, keepdims=True))
    counts = jnp.dot(jnp.ones((8, n), jnp.float32), onehot,
                     preferred_element_type=jnp.float32)
    lse_sum = jnp.dot(counts[0:1, :], lse_vec,
                      preferred_element_type=jnp.float32)

    tgt_sum = jnp.sum(jnp.where(colv == tgt_i, logits, 0.0), keepdims=True)
    losssum_ref[...] = (lse_sum - tgt_sum[0:1, 0:1])[None]


@jax.jit
def kernel(idx, targets, emb_padded):
    B, T = idx.shape
    V_pad = emb_padded.shape[1]
    N = B * T
    tile_n = _BLOCK_B * T
    num_tiles = B // _BLOCK_B

    cost = pl.CostEstimate(
        flops=2 * N * V_pad * V_pad,
        transcendentals=num_tiles * V_pad * V_pad,
        bytes_accessed=2 * N * 4 + V_pad * V_pad * 4 + N * V_pad * 4)
    logits, loss_sums = pl.pallas_call(
        _fused_kernel,
        out_shape=(
            jax.ShapeDtypeStruct((N, V_pad), jnp.float32),
            jax.ShapeDtypeStruct((num_tiles, 1, 1), jnp.float32),
        ),
        grid=(num_tiles,),
        in_specs=[
            pl.BlockSpec((_BLOCK_B, T), lambda i: (i, 0)),
            pl.BlockSpec((_BLOCK_B, T), lambda i: (i, 0)),
            pl.BlockSpec((V_pad, V_pad), lambda i: (0, 0)),
        ],
        out_specs=(
            pl.BlockSpec((tile_n, V_pad), lambda i: (i, 0)),
            pl.BlockSpec((1, 1, 1), lambda i: (i, 0, 0)),
        ),
        compiler_params=pltpu.CompilerParams(
            dimension_semantics=("parallel",),
            vmem_limit_bytes=64 * 1024 * 1024,
        ),
        cost_estimate=cost,
    )(idx, targets, emb_padded)

    loss = jnp.sum(loss_sums) / jnp.float32(N)
    return logits[:, :_V], loss


# speedup vs baseline: 1.0079x; 1.0079x over previous
import jax
import jax.numpy as jnp
from jax.experimental import pallas as pl
from jax.experimental.pallas import tpu as pltpu

_V = 200
_BLOCK_B = 32


def _fused_kernel(idx_ref, tgt_ref, emb_ref, sel_ref, tmask_ref,
                  logits_ref, losssum_ref):
    emb = emb_ref[...]
    rb, T = idx_ref.shape
    n = rb * T
    v_pad = emb.shape[1]

    idx_blk = idx_ref[...].astype(jnp.float32)
    tgt_blk = tgt_ref[...].astype(jnp.float32)

    sel = sel_ref[...]
    tmask = tmask_ref[...]

    rows_idx = jnp.dot(sel, idx_blk, preferred_element_type=jnp.float32)
    idx_i = jnp.sum(rows_idx * tmask,
                    axis=1, keepdims=True).astype(jnp.int32)
    rows_tgt = jnp.dot(sel, tgt_blk, preferred_element_type=jnp.float32)
    tgt_i = jnp.sum(rows_tgt * tmask,
                    axis=1, keepdims=True).astype(jnp.int32)

    colv = jax.lax.broadcasted_iota(jnp.int32, (n, v_pad), 1)
    onehot = (colv == idx_i).astype(jnp.float32)
    logits = jnp.dot(onehot, emb, preferred_element_type=jnp.float32)
    logits_ref[...] = logits

    m = jnp.max(emb, axis=1, keepdims=True)
    lse_vec = m + jnp.log(jnp.sum(jnp.exp(emb - m), axis=1, keepdims=True))
    counts = jnp.dot(jnp.ones((8, n), jnp.float32), onehot,
                     preferred_element_type=jnp.float32)
    lse_sum = jnp.dot(counts[0:1, :], lse_vec,
                      preferred_element_type=jnp.float32)

    tgt_sum = jnp.sum(jnp.where(colv == tgt_i, logits, 0.0), keepdims=True)
    losssum_ref[...] = (lse_sum - tgt_sum[0:1, 0:1])[None]


@jax.jit
def kernel(idx, targets, emb_padded):
    B, T = idx.shape
    V_pad = emb_padded.shape[1]
    N = B * T
    tile_n = _BLOCK_B * T
    num_tiles = B // _BLOCK_B

    n = tile_n
    row = jax.lax.broadcasted_iota(jnp.int32, (n, 1), 0)
    colb = jax.lax.broadcasted_iota(jnp.int32, (n, _BLOCK_B), 1)
    sel_c = ((row // T) == colb).astype(jnp.float32)
    colt = jax.lax.broadcasted_iota(jnp.int32, (n, T), 1)
    tmask_c = (colt == (row % T)).astype(jnp.float32)

    cost = pl.CostEstimate(
        flops=2 * N * V_pad * V_pad,
        transcendentals=num_tiles * V_pad * V_pad,
        bytes_accessed=2 * N * 4 + V_pad * V_pad * 4 + N * V_pad * 4)
    logits, loss_sums = pl.pallas_call(
        _fused_kernel,
        out_shape=(
            jax.ShapeDtypeStruct((N, V_pad), jnp.float32),
            jax.ShapeDtypeStruct((num_tiles, 1, 1), jnp.float32),
        ),
        grid=(num_tiles,),
        in_specs=[
            pl.BlockSpec((_BLOCK_B, T), lambda i: (i, 0)),
            pl.BlockSpec((_BLOCK_B, T), lambda i: (i, 0)),
            pl.BlockSpec((V_pad, V_pad), lambda i: (0, 0)),
            pl.BlockSpec((tile_n, _BLOCK_B), lambda i: (0, 0)),
            pl.BlockSpec((tile_n, T), lambda i: (0, 0)),
        ],
        out_specs=(
            pl.BlockSpec((tile_n, V_pad), lambda i: (i, 0)),
            pl.BlockSpec((1, 1, 1), lambda i: (i, 0, 0)),
        ),
        compiler_params=pltpu.CompilerParams(
            dimension_semantics=("parallel",),
            vmem_limit_bytes=64 * 1024 * 1024,
        ),
        cost_estimate=cost,
    )(idx, targets, emb_padded, sel_c, tmask_c)

    loss = jnp.sum(loss_sums) / jnp.float32(N)
    return logits[:, :_V], loss
